# baseline (speedup 1.0000x reference)
import functools

import jax
import jax.numpy as jnp
from jax import lax
from jax.experimental import pallas as pl
from jax.experimental.pallas import tpu as pltpu
from jax.experimental.pallas import tpu_sc as plsc

_INPUT_DIM = 100
_N_CONT = 74
_N_CAT = 26
_CAT_DIM = 100000
_EMB_DIM = 16
_BATCH = 16384
_OUT_DIM = _N_CONT + _N_CAT * _EMB_DIM

_NC, _NS = 2, 16
_NW = _NC * _NS
_SEG = _BATCH // _NW
_GROUPS = (2, 4, 6, 7, 7)


def _make_body(n_cat, with_cont, idx_row0):
    n_cont = _N_CONT if with_cont else 0

    def _body(xt_hbm, tt_hbm, out_hbm, idx_v, ci_v, cf_v, emb_v,
              sem, csem, ssem):
        wid = lax.axis_index("s") * _NC + lax.axis_index("c")
        seg = wid * _SEG
        pltpu.sync_copy(xt_hbm.at[pl.ds(idx_row0, n_cat), pl.ds(seg, _SEG)],
                        idx_v)
        if with_cont:
            cin = pltpu.async_copy(
                xt_hbm.at[pl.ds(0, _N_CONT), pl.ds(seg, _SEG)], ci_v, csem)

        def fire(c, buf):
            for e in range(_EMB_DIM):
                pltpu.async_copy(tt_hbm.at[c, e].at[idx_v.at[c]],
                                 emb_v.at[buf, e], sem)

        fire(0, 0)

        if with_cont:
            cin.wait()

            def conv_row(r, _):
                for k in range(_SEG // 16):
                    cf_v[r, pl.ds(16 * k, 16)] = (
                        ci_v[r, pl.ds(16 * k, 16)].astype(jnp.float32))
                return 0

            lax.fori_loop(0, _N_CONT, conv_row, 0)
            pltpu.sync_copy(cf_v,
                            out_hbm.at[pl.ds(0, _N_CONT), pl.ds(seg, _SEG)])

        def ship_wait(c):
            pltpu.make_async_copy(
                emb_v.at[c & 1],
                out_hbm.at[pl.ds(n_cont + _EMB_DIM * c, _EMB_DIM),
                           pl.ds(seg, _SEG)], ssem).wait()

        def table_step(c, _):
            @pl.when(c >= 1)
            def _():
                ship_wait(c - 1)

            @pl.when(c < n_cat - 1)
            def _():
                fire(c + 1, (c + 1) & 1)

            buf = c & 1
            for e in range(_EMB_DIM):
                pltpu.make_async_copy(tt_hbm.at[c, e].at[idx_v.at[c]],
                                      emb_v.at[buf, e], sem).wait()
            pltpu.async_copy(
                emb_v.at[buf],
                out_hbm.at[pl.ds(n_cont + _EMB_DIM * c, _EMB_DIM),
                           pl.ds(seg, _SEG)], ssem)
            return 0

        lax.fori_loop(0, n_cat, table_step, 0)
        ship_wait(n_cat - 1)

    return _body


def _group_call(n_cat, with_cont, idx_row0):
    n_out = n_cat * _EMB_DIM + (_N_CONT if with_cont else 0)
    return functools.partial(
        pl.kernel,
        out_type=jax.ShapeDtypeStruct((n_out, _BATCH), jnp.float32),
        mesh=plsc.VectorSubcoreMesh(core_axis_name="c", subcore_axis_name="s"),
        scratch_types=[
            pltpu.VMEM((n_cat, _SEG), jnp.int32),
            pltpu.VMEM((_N_CONT, _SEG), jnp.int32),
            pltpu.VMEM((_N_CONT, _SEG), jnp.float32),
            pltpu.VMEM((2, _EMB_DIM, _SEG), jnp.float32),
            pltpu.SemaphoreType.DMA,
            pltpu.SemaphoreType.DMA,
            pltpu.SemaphoreType.DMA,
        ],
        compiler_params=pltpu.CompilerParams(use_tc_tiling_on_sc=False),
    )(_make_body(n_cat, with_cont, idx_row0))


@jax.jit
def kernel(x, tables):
    xt = x.T
    tt = tables.transpose(0, 2, 1)
    outs = []
    a = 0
    for g, n_cat in enumerate(_GROUPS):
        with_cont = g == 0
        tt_g = tt[a:a + n_cat]
        outs.append(_group_call(n_cat, with_cont, _N_CONT + a)(xt, tt_g))
        a += n_cat
    out_t = jnp.concatenate(outs, axis=0)
    return out_t.T


# speedup vs baseline: 1.1528x; 1.1528x over previous
import functools

import jax
import jax.numpy as jnp
from jax import lax
from jax.experimental import pallas as pl
from jax.experimental.pallas import tpu as pltpu
from jax.experimental.pallas import tpu_sc as plsc

_INPUT_DIM = 100
_N_CONT = 74
_N_CAT = 26
_CAT_DIM = 100000
_EMB_DIM = 16
_BATCH = 16384
_OUT_DIM = _N_CONT + _N_CAT * _EMB_DIM

_NC, _NS = 2, 16
_NW = _NC * _NS
_SEG = _BATCH // _NW
_GROUPS = (7, 7, 6, 6)


def _make_body(n_cat, with_cont, idx_row0):
    n_cont = _N_CONT if with_cont else 0

    def _body(xt_hbm, tt_hbm, out_hbm, idx_v, ci_v, cf_v, emb_v,
              sem, csem, ssem):
        wid = lax.axis_index("s") * _NC + lax.axis_index("c")
        seg = wid * _SEG
        pltpu.sync_copy(xt_hbm.at[pl.ds(idx_row0, n_cat), pl.ds(seg, _SEG)],
                        idx_v)
        if with_cont:
            cin = pltpu.async_copy(
                xt_hbm.at[pl.ds(0, _N_CONT), pl.ds(seg, _SEG)], ci_v, csem)

        def fire(c, buf):
            for e in range(_EMB_DIM):
                pltpu.async_copy(tt_hbm.at[c, e].at[idx_v.at[c]],
                                 emb_v.at[buf, e], sem)

        fire(0, 0)

        if with_cont:
            cin.wait()

            def conv_row(r, _):
                for k in range(_SEG // 16):
                    cf_v[r, pl.ds(16 * k, 16)] = (
                        ci_v[r, pl.ds(16 * k, 16)].astype(jnp.float32))
                return 0

            lax.fori_loop(0, _N_CONT, conv_row, 0)
            pltpu.sync_copy(cf_v,
                            out_hbm.at[pl.ds(0, _N_CONT), pl.ds(seg, _SEG)])

        def ship_wait(c):
            pltpu.make_async_copy(
                emb_v.at[c & 1],
                out_hbm.at[pl.ds(n_cont + _EMB_DIM * c, _EMB_DIM),
                           pl.ds(seg, _SEG)], ssem).wait()

        def table_step(c, _):
            @pl.when(c >= 1)
            def _():
                ship_wait(c - 1)

            @pl.when(c < n_cat - 1)
            def _():
                fire(c + 1, (c + 1) & 1)

            buf = c & 1
            for e in range(_EMB_DIM):
                pltpu.make_async_copy(tt_hbm.at[c, e].at[idx_v.at[c]],
                                      emb_v.at[buf, e], sem).wait()
            pltpu.async_copy(
                emb_v.at[buf],
                out_hbm.at[pl.ds(n_cont + _EMB_DIM * c, _EMB_DIM),
                           pl.ds(seg, _SEG)], ssem)
            return 0

        lax.fori_loop(0, n_cat, table_step, 0)
        ship_wait(n_cat - 1)

    return _body


def _group_call(n_cat, with_cont, idx_row0):
    n_out = n_cat * _EMB_DIM + (_N_CONT if with_cont else 0)
    return functools.partial(
        pl.kernel,
        out_type=jax.ShapeDtypeStruct((n_out, _BATCH), jnp.float32),
        mesh=plsc.VectorSubcoreMesh(core_axis_name="c", subcore_axis_name="s"),
        scratch_types=[
            pltpu.VMEM((n_cat, _SEG), jnp.int32),
            pltpu.VMEM((_N_CONT, _SEG), jnp.int32),
            pltpu.VMEM((_N_CONT, _SEG), jnp.float32),
            pltpu.VMEM((2, _EMB_DIM, _SEG), jnp.float32),
            pltpu.SemaphoreType.DMA,
            pltpu.SemaphoreType.DMA,
            pltpu.SemaphoreType.DMA,
        ],
        compiler_params=pltpu.CompilerParams(use_tc_tiling_on_sc=False),
    )(_make_body(n_cat, with_cont, idx_row0))


@jax.jit
def kernel(x, tables):
    xt = x.T
    tt = tables.transpose(0, 2, 1)
    outs = []
    a = 0
    for g, n_cat in enumerate(_GROUPS):
        with_cont = g == 0
        tt_g = tt[a:a + n_cat]
        outs.append(_group_call(n_cat, with_cont, _N_CONT + a)(xt, tt_g))
        a += n_cat
    out_t = jnp.concatenate(outs, axis=0)
    return out_t.T
